# baseline (speedup 1.0000x reference)
import functools

import jax
import jax.numpy as jnp
from jax import lax
from jax.experimental import pallas as pl
from jax.experimental.pallas import tpu as pltpu
from jax.experimental.pallas import tpu_sc as plsc

_NC = 2
_NS = 16
_K = 64


def _mm2_body(x_ref, wa_ref, wb_ref, o_ref):
    x = x_ref[...]
    o_ref[0] = jnp.dot(x, wa_ref[...], preferred_element_type=jnp.float32)
    o_ref[1] = jnp.dot(x, wb_ref[...], preferred_element_type=jnp.float32)


def _matmul_split(x, wa, wb, blk):
    n, d = x.shape
    hh = wa.shape[1]
    return pl.pallas_call(
        _mm2_body,
        grid=(n // blk,),
        in_specs=[pl.BlockSpec((blk, d), lambda i: (i, 0)),
                  pl.BlockSpec((d, hh), lambda i: (0, 0)),
                  pl.BlockSpec((d, hh), lambda i: (0, 0))],
        out_specs=pl.BlockSpec((2, blk, hh), lambda i: (0, i, 0)),
        out_shape=jax.ShapeDtypeStruct((2, n, hh), jnp.float32),
    )(x, wa, wb)


def _bias_relu_mm2_body(p_ref, ba_ref, bb_ref, wa_ref, wb_ref, o_ref):
    ta = jnp.maximum(p_ref[0] + ba_ref[...], 0.0)
    tb = jnp.maximum(p_ref[1] + bb_ref[...], 0.0)
    o_ref[...] = (jnp.dot(ta, wa_ref[...], preferred_element_type=jnp.float32)
                  + jnp.dot(tb, wb_ref[...], preferred_element_type=jnp.float32))


def _bias_relu_matmul(p, ba, bb, wa, wb, blk):
    _, n, hh = p.shape
    cp = wa.shape[1]
    return pl.pallas_call(
        _bias_relu_mm2_body,
        grid=(n // blk,),
        in_specs=[pl.BlockSpec((2, blk, hh), lambda i: (0, i, 0)),
                  pl.BlockSpec((1, hh), lambda i: (0, 0)),
                  pl.BlockSpec((1, hh), lambda i: (0, 0)),
                  pl.BlockSpec((hh, cp), lambda i: (0, 0)),
                  pl.BlockSpec((hh, cp), lambda i: (0, 0))],
        out_specs=pl.BlockSpec((blk, cp), lambda i: (i, 0)),
        out_shape=jax.ShapeDtypeStruct((n, cp), jnp.float32),
    )(p, ba, bb, wa, wb)


def _make_softmax_body(c):
    def body(p_ref, b_ref, o_ref):
        v = p_ref[0] + p_ref[1] + b_ref[...]
        col = lax.broadcasted_iota(jnp.int32, v.shape, 1)
        valid = col < c
        m = jnp.max(jnp.where(valid, v, -jnp.inf), axis=1, keepdims=True)
        e = jnp.where(valid, jnp.exp(v - m), 0.0)
        o_ref[...] = (e / jnp.sum(e, axis=1, keepdims=True))[:, :c]
    return body


def _bias_softmax(parts, b, c, blk):
    _, n, cp = parts.shape
    return pl.pallas_call(
        _make_softmax_body(c),
        grid=(n // blk,),
        in_specs=[pl.BlockSpec((2, blk, cp), lambda i: (0, i, 0)),
                  pl.BlockSpec((1, cp), lambda i: (0, 0))],
        out_specs=pl.BlockSpec((blk, c), lambda i: (i, 0)),
        out_shape=jax.ShapeDtypeStruct((n, c), jnp.float32),
    )(parts, b)


def _make_agg(n, ep, hp, k, feature_split, nr=4):
    ntile = _NS if feature_split else _NC * _NS
    ept = ep // ntile
    nb = ept // k
    nq = nb // nr
    rpt = n // _NS
    nslice = hp // 16
    assert ep % ntile == 0 and ept % k == 0 and nb % nr == 0 and k % 16 == 0
    assert n % _NS == 0 and hp % 16 == 0 and nb >= 2 * nr

    mesh = plsc.VectorSubcoreMesh(core_axis_name="c", subcore_axis_name="s")
    h_shape = (_NC, n, hp) if feature_split else (n, hp)

    @functools.partial(
        pl.kernel,
        out_type=jax.ShapeDtypeStruct((_NC, n, hp), jnp.float32),
        mesh=mesh,
        compiler_params=pltpu.CompilerParams(use_tc_tiling_on_sc=False,
                                             needs_layout_passes=False),
        scratch_types=[
            pltpu.VMEM_SHARED((n, hp), jnp.float32),
            pltpu.VMEM((nb, k), jnp.int32),
            pltpu.VMEM((nb, k), jnp.int32),
            pltpu.VMEM((nb, k), jnp.float32),
            [pltpu.VMEM((k, hp), jnp.float32)] * nr,
            [pltpu.VMEM((k, hp), jnp.float32)] * nr,
            [pltpu.SemaphoreType.DMA] * nr,
            [pltpu.SemaphoreType.DMA] * nr,
        ],
    )
    def agg(h_hbm, ei3_hbm, w2_hbm, out_hbm,
            acc, src_v, dst_v, w_v, rows, scaled, sem_g, sem_s):
        c = lax.axis_index("c")
        s = lax.axis_index("s")
        if feature_split:
            h_my = h_hbm.at[c]
            b0 = s * nb
        else:
            h_my = h_hbm
            b0 = (c * _NS + s) * nb

        zero = jnp.zeros((16,), jnp.float32)
        for i in range(k):
            for j in range(nslice):
                rows[0][i, pl.ds(j * 16, 16)] = zero
        base = s * rpt
        full, tail = divmod(rpt, k)
        for i in range(full):
            pltpu.sync_copy(rows[0], acc.at[pl.ds(base + i * k, k)])
        if tail:
            pltpu.sync_copy(rows[0].at[pl.ds(0, tail)],
                            acc.at[pl.ds(base + full * k, tail)])

        pltpu.sync_copy(ei3_hbm.at[0, pl.ds(b0, nb)], src_v)
        pltpu.sync_copy(ei3_hbm.at[1, pl.ds(b0, nb)], dst_v)
        pltpu.sync_copy(w2_hbm.at[pl.ds(b0, nb)], w_v)
        plsc.subcore_barrier()

        idx16 = [jnp.full((16, 1), i, jnp.int32) for i in range(16)]
        dnums = lax.GatherDimensionNumbers(
            offset_dims=(), collapsed_slice_dims=(0,), start_index_map=(0,))

        def scale(src_buf, dst_buf, b):
            for g in range(k // 16):
                wrow = w_v[b, pl.ds(g * 16, 16)]
                for i in range(16):
                    wsplat = lax.gather(
                        wrow, idx16[i], dimension_numbers=dnums,
                        slice_sizes=(1,),
                        mode=lax.GatherScatterMode.PROMISE_IN_BOUNDS)
                    for j in range(nslice):
                        sl = pl.ds(j * 16, 16)
                        r = g * 16 + i
                        dst_buf[r, sl] = src_buf[r, sl] * wsplat

        def start_g(u, b):
            pltpu.async_copy(h_my.at[src_v.at[b]], rows[u], sem_g[u])

        def wait_g(u, b):
            pltpu.make_async_copy(h_my.at[src_v.at[b]], rows[u],
                                  sem_g[u]).wait()

        def start_s(u, b):
            pltpu.async_copy(scaled[u], acc.at[dst_v.at[b]], sem_s[u],
                             add=True)

        def wait_s(u, b):
            pltpu.make_async_copy(scaled[u], acc.at[dst_v.at[b]],
                                  sem_s[u]).wait()

        for u in range(nr):
            start_g(u, u)

        def body(q, carry):
            for u in range(nr):
                b = nr * q + u
                wait_g(u, b)

                @pl.when(b >= nr)
                def _():
                    wait_s(u, b - nr)

                scale(rows[u], scaled[u], b)
                start_s(u, b)

                @pl.when(b + nr < nb)
                def _():
                    start_g(u, b + nr)
            return carry

        lax.fori_loop(0, nq, body, 0)
        for b in range(nb - nr, nb):
            wait_s(b % nr, b)
        plsc.subcore_barrier()
        pltpu.sync_copy(acc.at[pl.ds(base, rpt)],
                        out_hbm.at[c, pl.ds(base, rpt)])

    return agg


def kernel(x, edge_index, edge_weight, W1, b1, W2, b2):
    n, d = x.shape
    e = edge_index.shape[1]
    h = W1.shape[1]
    c = W2.shape[1]
    hh = ((h + 31) // 32) * 32 // 2
    hp = 2 * hh
    cp = ((c + 15) // 16) * 16

    k1, k2 = 64, 128
    nw = _NC * _NS
    epw = 4 * k2 * 2
    ep = ((e + nw * epw - 1) // (nw * epw)) * (nw * epw)
    pad = ep - e
    fill = jnp.arange(pad, dtype=edge_index.dtype) % n
    ei = jnp.concatenate([edge_index, jnp.stack([fill, fill])], axis=1)
    ew = jnp.pad(edge_weight, (0, pad))

    w1p = jnp.pad(W1, ((0, 0), (0, hp - h)))
    b1p = jnp.pad(b1, (0, hp - h))[None, :]
    w2p = jnp.pad(W2, ((0, hp - h), (0, cp - c)))
    b2p = jnp.pad(b2, (0, cp - c))[None, :]

    h1 = _matmul_split(x, w1p[:, :hh], w1p[:, hh:], blk=1000)
    p1 = _make_agg(n, ep, hh, k1, feature_split=True, nr=2)(
        h1, ei.reshape(2, ep // k1, k1), ew.reshape(ep // k1, k1))
    h2 = _bias_relu_matmul(p1, b1p[:, :hh], b1p[:, hh:],
                           w2p[:hh], w2p[hh:], blk=1000)
    p2 = _make_agg(n, ep, cp, k2, feature_split=False, nr=2)(
        h2, ei.reshape(2, ep // k2, k2), ew.reshape(ep // k2, k2))
    return _bias_softmax(p2, b2p, c, blk=1000)


# speedup vs baseline: 1.0859x; 1.0859x over previous
import functools

import jax
import jax.numpy as jnp
from jax import lax
from jax.experimental import pallas as pl
from jax.experimental.pallas import tpu as pltpu
from jax.experimental.pallas import tpu_sc as plsc

_NC = 2
_NS = 16
_K = 64


def _mm2_body(x_ref, wa_ref, wb_ref, o_ref):
    x = x_ref[...]
    o_ref[0] = jnp.dot(x, wa_ref[...], preferred_element_type=jnp.float32)
    o_ref[1] = jnp.dot(x, wb_ref[...], preferred_element_type=jnp.float32)


def _matmul_split(x, wa, wb, blk):
    n, d = x.shape
    hh = wa.shape[1]
    return pl.pallas_call(
        _mm2_body,
        grid=(n // blk,),
        in_specs=[pl.BlockSpec((blk, d), lambda i: (i, 0)),
                  pl.BlockSpec((d, hh), lambda i: (0, 0)),
                  pl.BlockSpec((d, hh), lambda i: (0, 0))],
        out_specs=pl.BlockSpec((2, blk, hh), lambda i: (0, i, 0)),
        out_shape=jax.ShapeDtypeStruct((2, n, hh), jnp.float32),
    )(x, wa, wb)


def _bias_relu_mm2_body(p_ref, ba_ref, bb_ref, wa_ref, wb_ref, o_ref):
    ta = jnp.maximum(p_ref[0] + ba_ref[...], 0.0)
    tb = jnp.maximum(p_ref[1] + bb_ref[...], 0.0)
    o_ref[...] = (jnp.dot(ta, wa_ref[...], preferred_element_type=jnp.float32)
                  + jnp.dot(tb, wb_ref[...], preferred_element_type=jnp.float32))


def _bias_relu_matmul(p, ba, bb, wa, wb, blk):
    _, n, hh = p.shape
    cp = wa.shape[1]
    return pl.pallas_call(
        _bias_relu_mm2_body,
        grid=(n // blk,),
        in_specs=[pl.BlockSpec((2, blk, hh), lambda i: (0, i, 0)),
                  pl.BlockSpec((1, hh), lambda i: (0, 0)),
                  pl.BlockSpec((1, hh), lambda i: (0, 0)),
                  pl.BlockSpec((hh, cp), lambda i: (0, 0)),
                  pl.BlockSpec((hh, cp), lambda i: (0, 0))],
        out_specs=pl.BlockSpec((blk, cp), lambda i: (i, 0)),
        out_shape=jax.ShapeDtypeStruct((n, cp), jnp.float32),
    )(p, ba, bb, wa, wb)


def _make_softmax_body(c):
    def body(p_ref, b_ref, o_ref):
        v = p_ref[0] + p_ref[1] + b_ref[...]
        col = lax.broadcasted_iota(jnp.int32, v.shape, 1)
        valid = col < c
        m = jnp.max(jnp.where(valid, v, -jnp.inf), axis=1, keepdims=True)
        e = jnp.where(valid, jnp.exp(v - m), 0.0)
        o_ref[...] = (e / jnp.sum(e, axis=1, keepdims=True))[:, :c]
    return body


def _bias_softmax(parts, b, c, blk):
    _, n, cp = parts.shape
    return pl.pallas_call(
        _make_softmax_body(c),
        grid=(n // blk,),
        in_specs=[pl.BlockSpec((2, blk, cp), lambda i: (0, i, 0)),
                  pl.BlockSpec((1, cp), lambda i: (0, 0))],
        out_specs=pl.BlockSpec((blk, c), lambda i: (i, 0)),
        out_shape=jax.ShapeDtypeStruct((n, c), jnp.float32),
    )(parts, b)


def _make_agg(n, ep, hp, k, feature_split, nr=4):
    ntile = _NS if feature_split else _NC * _NS
    ept = ep // ntile
    nb = ept // k
    nq = nb // nr
    rpt = n // _NS
    nslice = hp // 16
    assert ep % ntile == 0 and ept % k == 0 and nb % nr == 0 and k % 16 == 0
    assert n % _NS == 0 and hp % 16 == 0 and nb >= 2 * nr

    mesh = plsc.VectorSubcoreMesh(core_axis_name="c", subcore_axis_name="s")
    h_shape = (_NC, n, hp) if feature_split else (n, hp)

    @functools.partial(
        pl.kernel,
        out_type=jax.ShapeDtypeStruct((_NC, n, hp), jnp.float32),
        mesh=mesh,
        compiler_params=pltpu.CompilerParams(use_tc_tiling_on_sc=False,
                                             needs_layout_passes=False),
        scratch_types=[
            pltpu.VMEM_SHARED((n, hp), jnp.float32),
            pltpu.VMEM((nb, k), jnp.int32),
            pltpu.VMEM((nb, k), jnp.int32),
            pltpu.VMEM((nb, k), jnp.float32),
            [pltpu.VMEM((k, hp), jnp.float32)] * nr,
            [pltpu.VMEM((k, hp), jnp.float32)] * nr,
            [pltpu.SemaphoreType.DMA] * nr,
            [pltpu.SemaphoreType.DMA] * nr,
        ],
    )
    def agg(h_hbm, ei3_hbm, w2_hbm, out_hbm,
            acc, src_v, dst_v, w_v, rows, scaled, sem_g, sem_s):
        c = lax.axis_index("c")
        s = lax.axis_index("s")
        if feature_split:
            h_my = h_hbm.at[c]
            b0 = s * nb
        else:
            h_my = h_hbm
            b0 = (c * _NS + s) * nb

        zero = jnp.zeros((16,), jnp.float32)
        for i in range(k):
            for j in range(nslice):
                rows[0][i, pl.ds(j * 16, 16)] = zero
        base = s * rpt
        full, tail = divmod(rpt, k)
        for i in range(full):
            pltpu.sync_copy(rows[0], acc.at[pl.ds(base + i * k, k)])
        if tail:
            pltpu.sync_copy(rows[0].at[pl.ds(0, tail)],
                            acc.at[pl.ds(base + full * k, tail)])

        pltpu.sync_copy(ei3_hbm.at[0, pl.ds(b0, nb)], src_v)
        pltpu.sync_copy(ei3_hbm.at[1, pl.ds(b0, nb)], dst_v)
        pltpu.sync_copy(w2_hbm.at[pl.ds(b0, nb)], w_v)
        plsc.subcore_barrier()

        idx16 = [jnp.full((16, 1), i, jnp.int32) for i in range(16)]
        dnums = lax.GatherDimensionNumbers(
            offset_dims=(), collapsed_slice_dims=(0,), start_index_map=(0,))

        def scale(src_buf, dst_buf, b):
            for g in range(k // 16):
                wrow = w_v[b, pl.ds(g * 16, 16)]
                for i in range(16):
                    wsplat = lax.gather(
                        wrow, idx16[i], dimension_numbers=dnums,
                        slice_sizes=(1,),
                        mode=lax.GatherScatterMode.PROMISE_IN_BOUNDS)
                    for j in range(nslice):
                        sl = pl.ds(j * 16, 16)
                        r = g * 16 + i
                        dst_buf[r, sl] = src_buf[r, sl] * wsplat

        def start_g(u, b):
            pltpu.async_copy(h_my.at[src_v.at[b]], rows[u], sem_g[u])

        def wait_g(u, b):
            pltpu.make_async_copy(h_my.at[src_v.at[b]], rows[u],
                                  sem_g[u]).wait()

        def start_s(u, b):
            pltpu.async_copy(scaled[u], acc.at[dst_v.at[b]], sem_s[u],
                             add=True)

        def wait_s(u, b):
            pltpu.make_async_copy(scaled[u], acc.at[dst_v.at[b]],
                                  sem_s[u]).wait()

        for u in range(nr):
            start_g(u, u)

        def body(q, carry):
            for u in range(nr):
                b = nr * q + u
                wait_g(u, b)

                @pl.when(b >= nr)
                def _():
                    wait_s(u, b - nr)

                scale(rows[u], scaled[u], b)
                start_s(u, b)

                @pl.when(b + nr < nb)
                def _():
                    start_g(u, b + nr)
            return carry

        lax.fori_loop(0, nq, body, 0)
        for b in range(nb - nr, nb):
            wait_s(b % nr, b)
        plsc.subcore_barrier()
        pltpu.sync_copy(acc.at[pl.ds(base, rpt)],
                        out_hbm.at[c, pl.ds(base, rpt)])

    return agg


def kernel(x, edge_index, edge_weight, W1, b1, W2, b2):
    n, d = x.shape
    e = edge_index.shape[1]
    h = W1.shape[1]
    c = W2.shape[1]
    hh = ((h + 31) // 32) * 32 // 2
    hp = 2 * hh
    cp = ((c + 15) // 16) * 16

    k1, k2 = 32, 64
    nw = _NC * _NS
    epw = 4 * k2 * 2
    ep = ((e + nw * epw - 1) // (nw * epw)) * (nw * epw)
    pad = ep - e
    fill = jnp.arange(pad, dtype=edge_index.dtype) % n
    ei = jnp.concatenate([edge_index, jnp.stack([fill, fill])], axis=1)
    ew = jnp.pad(edge_weight, (0, pad))

    w1p = jnp.pad(W1, ((0, 0), (0, hp - h)))
    b1p = jnp.pad(b1, (0, hp - h))[None, :]
    w2p = jnp.pad(W2, ((0, hp - h), (0, cp - c)))
    b2p = jnp.pad(b2, (0, cp - c))[None, :]

    h1 = _matmul_split(x, w1p[:, :hh], w1p[:, hh:], blk=1000)
    p1 = _make_agg(n, ep, hh, k1, feature_split=True, nr=4)(
        h1, ei.reshape(2, ep // k1, k1), ew.reshape(ep // k1, k1))
    h2 = _bias_relu_matmul(p1, b1p[:, :hh], b1p[:, hh:],
                           w2p[:hh], w2p[hh:], blk=1000)
    p2 = _make_agg(n, ep, cp, k2, feature_split=False, nr=4)(
        h2, ei.reshape(2, ep // k2, k2), ew.reshape(ep // k2, k2))
    return _bias_softmax(p2, b2p, c, blk=1000)
